# baseline (speedup 1.0000x reference)
import functools

import jax
import jax.numpy as jnp
from jax import lax
from jax.experimental import pallas as pl
from jax.experimental.pallas import tpu as pltpu
from jax.experimental.pallas import tpu_sc as plsc

_NC = 2
_NS = 16
_NW = _NC * _NS


def _tc_matmul_bf16(x, W, b_half):
    n, d_in = x.shape
    d_out = W.shape[1]

    def body(x_ref, w_ref, b_ref, o_ref):
        o_ref[...] = (
            jnp.dot(x_ref[...], w_ref[...], preferred_element_type=jnp.float32)
            + b_ref[...]
        ).astype(jnp.bfloat16)

    return pl.pallas_call(
        body,
        out_shape=jax.ShapeDtypeStruct((n, d_out), jnp.bfloat16),
    )(x, W, b_half)


def _make_sc_gather_add(E, D, C, NBUF):
    epw = E // _NW
    nchunks = epw // C
    nouter = nchunks // NBUF
    main = nouter * NBUF
    ntail = nchunks - main
    assert ntail == 2 and NBUF >= 4 and epw % C == 0 and C % 8 == 0 and C <= 128
    Dw = D // 2
    mesh = plsc.VectorSubcoreMesh(core_axis_name="c", subcore_axis_name="s")

    @functools.partial(
        pl.kernel,
        mesh=mesh,
        compiler_params=pltpu.CompilerParams(
            needs_layout_passes=False, use_tc_tiling_on_sc=False),
        out_type=jax.ShapeDtypeStruct((E, D), jnp.float32),
        scratch_types=[
            pltpu.VMEM((epw,), jnp.int32),
            pltpu.VMEM((epw,), jnp.int32),
            pltpu.VMEM((NBUF, C, Dw), jnp.int32),
            pltpu.VMEM((NBUF, C, Dw), jnp.int32),
            pltpu.VMEM((NBUF, C, D), jnp.float32),
            pltpu.SemaphoreType.DMA((NBUF,)),
            pltpu.SemaphoreType.DMA((NBUF,)),
        ],
    )
    def sc_fn(y_hbm, row_hbm, col_hbm, out_hbm,
              idxr, idxc, bufa, bufb, bufo, gsem, wsem):
        wid = lax.axis_index("s") * _NC + lax.axis_index("c")
        base = wid * epw

        pltpu.sync_copy(row_hbm.at[pl.ds(base, epw)], idxr)
        pltpu.sync_copy(col_hbm.at[pl.ds(base, epw)], idxc)

        ii = lax.iota(jnp.int32, 16)
        even = ii * 2
        odd = even + 1

        def fire_gather(j, s):
            o = j * C
            pltpu.async_copy(y_hbm.at[idxr.at[pl.ds(o, C)]], bufa.at[s], gsem.at[s])
            pltpu.async_copy(y_hbm.at[idxc.at[pl.ds(o, C)]], bufb.at[s], gsem.at[s])

        def wait_gather(j, s):
            o = j * C
            pltpu.make_async_copy(
                y_hbm.at[idxr.at[pl.ds(o, C)]], bufa.at[s], gsem.at[s]).wait()
            pltpu.make_async_copy(
                y_hbm.at[idxc.at[pl.ds(o, C)]], bufb.at[s], gsem.at[s]).wait()

        def fire_write(j, s):
            o = base + j * C
            pltpu.async_copy(bufo.at[s], out_hbm.at[pl.ds(o, C)], wsem.at[s])

        def wait_write(j, s):
            o = base + j * C
            pltpu.make_async_copy(
                bufo.at[s], out_hbm.at[pl.ds(o, C)], wsem.at[s]).wait()

        himask = jnp.full((16,), -0x10000, dtype=jnp.int32)

        def widen_lo(v):
            return plsc.bitcast(v << 16, jnp.float32)

        def widen_hi(v):
            return plsc.bitcast(v & himask, jnp.float32)

        def do_add(s):
            def add_body(e, c2):
                orow = bufo.at[s, e]
                for g in range(Dw // 16):
                    sl = pl.ds(g * 16, 16)
                    va = bufa[s, e, sl]
                    vb = bufb[s, e, sl]
                    lo = widen_lo(va) + widen_lo(vb)
                    hi = widen_hi(va) + widen_hi(vb)
                    plsc.store_scatter(orow, [even + g * 32], lo)
                    plsc.store_scatter(orow, [odd + g * 32], hi)
                return c2

            lax.fori_loop(0, C, add_body, 0, unroll=False)

        fire_gather(0, 0)
        fire_gather(1, 1)

        def outer(jj, carry):
            for s in range(NBUF):
                j = jj * NBUF + s
                if s < 2:
                    @pl.when(jj >= 1)
                    def _w():
                        wait_write(j - 2, (s + 2) % NBUF)
                else:
                    wait_write(j - 2, s - 2)
                fire_gather(j + 2, (s + 2) % NBUF)
                wait_gather(j, s)
                do_add(s)
                fire_write(j, s)
            return carry

        lax.fori_loop(0, nouter, outer, 0, unroll=False)

        for t in range(ntail):
            j = main + t
            wait_write(j - 2, (t + 2) % NBUF)
            wait_gather(j, t)
            do_add(t)
            fire_write(j, t)
        for t in range(ntail):
            wait_write(main + t, t)

    return sc_fn


def kernel(x, edge_index, W, b):
    n, d_in = x.shape
    d_out = W.shape[1]
    E = edge_index.shape[1]

    b_half = (0.5 * b).reshape(1, d_out).astype(jnp.float32)
    y = _tc_matmul_bf16(x, W, b_half)
    y32 = lax.bitcast_convert_type(y.reshape(n, d_out // 2, 2), jnp.int32)

    sc_fn = _make_sc_gather_add(E, d_out, C=40, NBUF=4)
    row = edge_index[0]
    col = edge_index[1]
    return sc_fn(y32, row, col)


# speedup vs baseline: 1.0248x; 1.0248x over previous
import functools

import jax
import jax.numpy as jnp
from jax import lax
from jax.experimental import pallas as pl
from jax.experimental.pallas import tpu as pltpu
from jax.experimental.pallas import tpu_sc as plsc

_NC = 2
_NS = 16
_NW = _NC * _NS


def _tc_matmul_bf16(x, W, b_half):
    n, d_in = x.shape
    d_out = W.shape[1]

    def body(x_ref, w_ref, b_ref, o_ref):
        o_ref[...] = (
            jnp.dot(x_ref[...], w_ref[...], preferred_element_type=jnp.float32)
            + b_ref[...]
        ).astype(jnp.bfloat16)

    return pl.pallas_call(
        body,
        out_shape=jax.ShapeDtypeStruct((n, d_out), jnp.bfloat16),
    )(x, W, b_half)


def _make_sc_gather_add(E, D, C, NBUF):
    epw = E // _NW
    nchunks = epw // C
    nouter = nchunks // NBUF
    main = nouter * NBUF
    ntail = nchunks - main
    assert ntail == 2 and NBUF >= 4 and epw % C == 0 and C % 8 == 0 and C <= 128
    Dw = D // 2
    mesh = plsc.VectorSubcoreMesh(core_axis_name="c", subcore_axis_name="s")

    @functools.partial(
        pl.kernel,
        mesh=mesh,
        compiler_params=pltpu.CompilerParams(
            needs_layout_passes=False, use_tc_tiling_on_sc=False),
        out_type=jax.ShapeDtypeStruct((E, D), jnp.float32),
        scratch_types=[
            pltpu.VMEM((epw,), jnp.int32),
            pltpu.VMEM((epw,), jnp.int32),
            pltpu.VMEM((NBUF, C, Dw), jnp.int32),
            pltpu.VMEM((NBUF, C, Dw), jnp.int32),
            pltpu.VMEM((NBUF, C, D), jnp.float32),
            pltpu.SemaphoreType.DMA((NBUF,)),
            pltpu.SemaphoreType.DMA((NBUF,)),
        ],
    )
    def sc_fn(y_hbm, row_hbm, col_hbm, out_hbm,
              idxr, idxc, bufa, bufb, bufo, gsem, wsem):
        wid = lax.axis_index("s") * _NC + lax.axis_index("c")
        base = wid * epw

        pltpu.sync_copy(row_hbm.at[pl.ds(base, epw)], idxr)
        pltpu.sync_copy(col_hbm.at[pl.ds(base, epw)], idxc)

        def fire_gather(j, s):
            o = j * C
            pltpu.async_copy(y_hbm.at[idxr.at[pl.ds(o, C)]], bufa.at[s], gsem.at[s])
            pltpu.async_copy(y_hbm.at[idxc.at[pl.ds(o, C)]], bufb.at[s], gsem.at[s])

        def wait_gather(j, s):
            o = j * C
            pltpu.make_async_copy(
                y_hbm.at[idxr.at[pl.ds(o, C)]], bufa.at[s], gsem.at[s]).wait()
            pltpu.make_async_copy(
                y_hbm.at[idxc.at[pl.ds(o, C)]], bufb.at[s], gsem.at[s]).wait()

        def fire_write(j, s):
            o = base + j * C
            pltpu.async_copy(bufo.at[s], out_hbm.at[pl.ds(o, C)], wsem.at[s])

        def wait_write(j, s):
            o = base + j * C
            pltpu.make_async_copy(
                bufo.at[s], out_hbm.at[pl.ds(o, C)], wsem.at[s]).wait()

        himask = jnp.full((16,), -0x10000, dtype=jnp.int32)

        def widen_lo(v):
            return plsc.bitcast(v << 16, jnp.float32)

        def widen_hi(v):
            return plsc.bitcast(v & himask, jnp.float32)

        def do_add(s):
            def add_body(e, c2):
                for g in range(Dw // 16):
                    sl = pl.ds(g * 16, 16)
                    va = bufa[s, e, sl]
                    vb = bufb[s, e, sl]
                    bufo[s, e, pl.ds(g * 32, 16)] = widen_lo(va) + widen_lo(vb)
                    bufo[s, e, pl.ds(g * 32 + 16, 16)] = widen_hi(va) + widen_hi(vb)
                return c2

            lax.fori_loop(0, C, add_body, 0, unroll=False)

        fire_gather(0, 0)
        fire_gather(1, 1)

        def outer(jj, carry):
            for s in range(NBUF):
                j = jj * NBUF + s
                if s < 2:
                    @pl.when(jj >= 1)
                    def _w():
                        wait_write(j - 2, (s + 2) % NBUF)
                else:
                    wait_write(j - 2, s - 2)
                fire_gather(j + 2, (s + 2) % NBUF)
                wait_gather(j, s)
                do_add(s)
                fire_write(j, s)
            return carry

        lax.fori_loop(0, nouter, outer, 0, unroll=False)

        for t in range(ntail):
            j = main + t
            wait_write(j - 2, (t + 2) % NBUF)
            wait_gather(j, t)
            do_add(t)
            fire_write(j, t)
        for t in range(ntail):
            wait_write(main + t, t)

    return sc_fn


def kernel(x, edge_index, W, b):
    n, d_in = x.shape
    d_out = W.shape[1]
    E = edge_index.shape[1]

    p = jnp.arange(d_out)
    perm = 32 * (p // 32) + (p % 32) // 2 + 16 * (p % 2)
    b_half = (0.5 * b)[perm].reshape(1, d_out).astype(jnp.float32)
    y = _tc_matmul_bf16(x, W[:, perm], b_half)
    y32 = lax.bitcast_convert_type(y.reshape(n, d_out // 2, 2), jnp.int32)

    sc_fn = _make_sc_gather_add(E, d_out, C=40, NBUF=4)
    row = edge_index[0]
    col = edge_index[1]
    return sc_fn(y32, row, col)


# speedup vs baseline: 1.5142x; 1.4776x over previous
import functools

import jax
import jax.numpy as jnp
from jax import lax
from jax.experimental import pallas as pl
from jax.experimental.pallas import tpu as pltpu
from jax.experimental.pallas import tpu_sc as plsc

_NC = 2
_NS = 16
_NW = _NC * _NS


def _tc_matmul(x, W, b_half):
    n, d_in = x.shape
    d_out = W.shape[1]

    def body(x_ref, w_ref, b_ref, o_ref):
        o_ref[...] = (
            jnp.dot(x_ref[...], w_ref[...], preferred_element_type=jnp.float32)
            + b_ref[...]
        )

    return pl.pallas_call(
        body,
        out_shape=jax.ShapeDtypeStruct((n, d_out), jnp.float32),
    )(x, W, b_half)


def _make_sc_gather_add(E, D, C, NBUF):
    epw = E // _NW
    nchunks = epw // C
    nouter = nchunks // NBUF
    main = nouter * NBUF
    ntail = nchunks - main
    assert ntail == 2 and NBUF >= 4 and epw % C == 0 and C % 8 == 0 and C <= 128
    mesh = plsc.VectorSubcoreMesh(core_axis_name="c", subcore_axis_name="s")

    @functools.partial(
        pl.kernel,
        mesh=mesh,
        compiler_params=pltpu.CompilerParams(
            needs_layout_passes=False, use_tc_tiling_on_sc=False),
        out_type=jax.ShapeDtypeStruct((E, D), jnp.float32),
        scratch_types=[
            pltpu.VMEM((epw,), jnp.int32),
            pltpu.VMEM((epw,), jnp.int32),
            pltpu.VMEM((NBUF, C, D), jnp.float32),
            pltpu.VMEM((NBUF, C, D), jnp.float32),
            pltpu.SemaphoreType.DMA((NBUF,)),
            pltpu.SemaphoreType.DMA((NBUF,)),
        ],
    )
    def sc_fn(y_hbm, row_hbm, col_hbm, out_hbm, idxr, idxc, bufa, bufb, gsem, wsem):
        wid = lax.axis_index("s") * _NC + lax.axis_index("c")
        base = wid * epw

        pltpu.sync_copy(row_hbm.at[pl.ds(base, epw)], idxr)
        pltpu.sync_copy(col_hbm.at[pl.ds(base, epw)], idxc)

        def fire_gather(j, s):
            o = j * C
            pltpu.async_copy(y_hbm.at[idxr.at[pl.ds(o, C)]], bufa.at[s], gsem.at[s])
            pltpu.async_copy(y_hbm.at[idxc.at[pl.ds(o, C)]], bufb.at[s], gsem.at[s])

        def wait_gather(j, s):
            o = j * C
            pltpu.make_async_copy(
                y_hbm.at[idxr.at[pl.ds(o, C)]], bufa.at[s], gsem.at[s]).wait()
            pltpu.make_async_copy(
                y_hbm.at[idxc.at[pl.ds(o, C)]], bufb.at[s], gsem.at[s]).wait()

        def fire_write(j, s):
            o = base + j * C
            pltpu.async_copy(bufa.at[s], out_hbm.at[pl.ds(o, C)], wsem.at[s])

        def wait_write(j, s):
            o = base + j * C
            pltpu.make_async_copy(
                bufa.at[s], out_hbm.at[pl.ds(o, C)], wsem.at[s]).wait()

        def do_add(s):
            def add_body(e, c2):
                for k in range(D // 16):
                    sl = pl.ds(k * 16, 16)
                    plsc.addupdate(bufa.at[s, e, sl], bufb[s, e, sl])
                return c2

            lax.fori_loop(0, C, add_body, 0, unroll=False)

        fire_gather(0, 0)
        fire_gather(1, 1)

        def outer(jj, carry):
            for s in range(NBUF):
                j = jj * NBUF + s
                if s < 2:
                    @pl.when(jj >= 1)
                    def _w():
                        wait_write(j - 2, (s + 2) % NBUF)
                else:
                    wait_write(j - 2, s - 2)
                fire_gather(j + 2, (s + 2) % NBUF)
                wait_gather(j, s)
                do_add(s)
                fire_write(j, s)
            return carry

        lax.fori_loop(0, nouter, outer, 0, unroll=False)

        for t in range(ntail):
            j = main + t
            wait_write(j - 2, (t + 2) % NBUF)
            wait_gather(j, t)
            do_add(t)
            fire_write(j, t)
        for t in range(ntail):
            wait_write(main + t, t)

    return sc_fn


def kernel(x, edge_index, W, b):
    n, d_in = x.shape
    d_out = W.shape[1]
    E = edge_index.shape[1]

    b_half = (0.5 * b).reshape(1, d_out).astype(jnp.float32)
    y = _tc_matmul(x, W, b_half)

    sc_fn = _make_sc_gather_add(E, d_out, C=40, NBUF=4)
    row = edge_index[0]
    col = edge_index[1]
    return sc_fn(y, row, col)


# speedup vs baseline: 1.6383x; 1.0819x over previous
import functools

import jax
import jax.numpy as jnp
from jax import lax
from jax.experimental import pallas as pl
from jax.experimental.pallas import tpu as pltpu
from jax.experimental.pallas import tpu_sc as plsc

_NC = 2
_NS = 16
_NW = _NC * _NS


def _tc_matmul_bf16(x, W, b_half):
    n, d_in = x.shape
    d_out = W.shape[1]

    def body(x_ref, w_ref, b_ref, o_ref):
        o_ref[...] = (
            jnp.dot(x_ref[...], w_ref[...], preferred_element_type=jnp.float32)
            + b_ref[...]
        ).astype(jnp.bfloat16)

    return pl.pallas_call(
        body,
        out_shape=jax.ShapeDtypeStruct((n, d_out), jnp.bfloat16),
    )(x, W, b_half)


def _make_sc_gather_add(E, D, C, NBUF):
    epw = E // _NW
    nchunks = epw // C
    nouter = nchunks // NBUF
    main = nouter * NBUF
    ntail = nchunks - main
    assert ntail == 2 and NBUF >= 4 and epw % C == 0 and C % 8 == 0 and C <= 128
    Dw = D // 2
    mesh = plsc.VectorSubcoreMesh(core_axis_name="c", subcore_axis_name="s")

    @functools.partial(
        pl.kernel,
        mesh=mesh,
        compiler_params=pltpu.CompilerParams(
            needs_layout_passes=False, use_tc_tiling_on_sc=False),
        out_type=jax.ShapeDtypeStruct((E, D), jnp.float32),
        scratch_types=[
            pltpu.VMEM((epw,), jnp.int32),
            pltpu.VMEM((epw,), jnp.int32),
            pltpu.VMEM((NBUF, C, Dw), jnp.int32),
            pltpu.VMEM((NBUF, C, Dw), jnp.int32),
            pltpu.VMEM((NBUF, C, D), jnp.float32),
            pltpu.SemaphoreType.DMA((NBUF,)),
            pltpu.SemaphoreType.DMA((NBUF,)),
        ],
    )
    def sc_fn(y_hbm, row_hbm, col_hbm, out_hbm,
              idxr, idxc, bufa, bufb, bufo, gsem, wsem):
        wid = lax.axis_index("s") * _NC + lax.axis_index("c")
        base = wid * epw

        pltpu.sync_copy(row_hbm.at[pl.ds(base, epw)], idxr)
        pltpu.sync_copy(col_hbm.at[pl.ds(base, epw)], idxc)

        def fire_gather(j, s):
            o = j * C
            pltpu.async_copy(y_hbm.at[idxr.at[pl.ds(o, C)]], bufa.at[s], gsem.at[s])
            pltpu.async_copy(y_hbm.at[idxc.at[pl.ds(o, C)]], bufb.at[s], gsem.at[s])

        def wait_gather(j, s):
            o = j * C
            pltpu.make_async_copy(
                y_hbm.at[idxr.at[pl.ds(o, C)]], bufa.at[s], gsem.at[s]).wait()
            pltpu.make_async_copy(
                y_hbm.at[idxc.at[pl.ds(o, C)]], bufb.at[s], gsem.at[s]).wait()

        def fire_write(j, s):
            o = base + j * C
            pltpu.async_copy(bufo.at[s], out_hbm.at[pl.ds(o, C)], wsem.at[s])

        def wait_write(j, s):
            o = base + j * C
            pltpu.make_async_copy(
                bufo.at[s], out_hbm.at[pl.ds(o, C)], wsem.at[s]).wait()

        himask = jnp.full((16,), -0x10000, dtype=jnp.int32)

        def widen_lo(v):
            return plsc.bitcast(v << 16, jnp.float32)

        def widen_hi(v):
            return plsc.bitcast(v & himask, jnp.float32)

        def do_add(s):
            @plsc.parallel_loop(0, C, unroll=2)
            def _add_body(e):
                for g in range(Dw // 16):
                    sl = pl.ds(g * 16, 16)
                    va = bufa[s, e, sl]
                    vb = bufb[s, e, sl]
                    bufo[s, e, pl.ds(g * 32, 16)] = widen_lo(va) + widen_lo(vb)
                    bufo[s, e, pl.ds(g * 32 + 16, 16)] = widen_hi(va) + widen_hi(vb)

        fire_gather(0, 0)
        fire_gather(1, 1)

        def outer(jj, carry):
            for s in range(NBUF):
                j = jj * NBUF + s
                if s < 2:
                    @pl.when(jj >= 1)
                    def _w():
                        wait_write(j - 2, (s + 2) % NBUF)
                else:
                    wait_write(j - 2, s - 2)
                fire_gather(j + 2, (s + 2) % NBUF)
                wait_gather(j, s)
                do_add(s)
                fire_write(j, s)
            return carry

        lax.fori_loop(0, nouter, outer, 0, unroll=False)

        for t in range(ntail):
            j = main + t
            wait_write(j - 2, (t + 2) % NBUF)
            wait_gather(j, t)
            do_add(t)
            fire_write(j, t)
        for t in range(ntail):
            wait_write(main + t, t)

    return sc_fn


def kernel(x, edge_index, W, b):
    n, d_in = x.shape
    d_out = W.shape[1]
    E = edge_index.shape[1]

    p = jnp.arange(d_out)
    perm = 32 * (p // 32) + (p % 32) // 2 + 16 * (p % 2)
    b_half = (0.5 * b)[perm].reshape(1, d_out).astype(jnp.float32)
    y = _tc_matmul_bf16(x, W[:, perm], b_half)
    y32 = lax.bitcast_convert_type(y.reshape(n, d_out // 2, 2), jnp.int32)

    sc_fn = _make_sc_gather_add(E, d_out, C=40, NBUF=4)
    row = edge_index[0]
    col = edge_index[1]
    return sc_fn(y32, row, col)


# speedup vs baseline: 1.6390x; 1.0004x over previous
import functools

import jax
import jax.numpy as jnp
from jax import lax
from jax.experimental import pallas as pl
from jax.experimental.pallas import tpu as pltpu
from jax.experimental.pallas import tpu_sc as plsc

_NC = 2
_NS = 16
_NW = _NC * _NS


def _tc_matmul_bf16(x, W, b_half):
    n, d_in = x.shape
    d_out = W.shape[1]

    def body(x_ref, w_ref, b_ref, o_ref):
        o_ref[...] = (
            jnp.dot(x_ref[...], w_ref[...], preferred_element_type=jnp.float32)
            + b_ref[...]
        ).astype(jnp.bfloat16)

    return pl.pallas_call(
        body,
        out_shape=jax.ShapeDtypeStruct((n, d_out), jnp.bfloat16),
    )(x, W, b_half)


def _make_sc_gather_add(E, D, C, NBUF):
    epw = E // _NW
    nchunks = epw // C
    nouter = nchunks // NBUF
    main = nouter * NBUF
    ntail = nchunks - main
    assert ntail == 2 and NBUF >= 4 and epw % C == 0 and C % 8 == 0 and C <= 128
    Dw = D // 2
    mesh = plsc.VectorSubcoreMesh(core_axis_name="c", subcore_axis_name="s")

    @functools.partial(
        pl.kernel,
        mesh=mesh,
        compiler_params=pltpu.CompilerParams(
            needs_layout_passes=False, use_tc_tiling_on_sc=False),
        out_type=jax.ShapeDtypeStruct((E, D), jnp.float32),
        scratch_types=[
            pltpu.VMEM((epw,), jnp.int32),
            pltpu.VMEM((epw,), jnp.int32),
            pltpu.VMEM((NBUF, C, Dw), jnp.int32),
            pltpu.VMEM((NBUF, C, Dw), jnp.int32),
            pltpu.VMEM((NBUF, C, D), jnp.float32),
            pltpu.SemaphoreType.DMA((NBUF,)),
            pltpu.SemaphoreType.DMA((NBUF,)),
        ],
    )
    def sc_fn(y_hbm, row_hbm, col_hbm, out_hbm,
              idxr, idxc, bufa, bufb, bufo, gsem, wsem):
        wid = lax.axis_index("s") * _NC + lax.axis_index("c")
        base = wid * epw

        pltpu.sync_copy(row_hbm.at[pl.ds(base, epw)], idxr)
        pltpu.sync_copy(col_hbm.at[pl.ds(base, epw)], idxc)

        def fire_gather(j, s):
            o = j * C
            pltpu.async_copy(y_hbm.at[idxr.at[pl.ds(o, C)]], bufa.at[s], gsem.at[s])
            pltpu.async_copy(y_hbm.at[idxc.at[pl.ds(o, C)]], bufb.at[s], gsem.at[s])

        def wait_gather(j, s):
            o = j * C
            pltpu.make_async_copy(
                y_hbm.at[idxr.at[pl.ds(o, C)]], bufa.at[s], gsem.at[s]).wait()
            pltpu.make_async_copy(
                y_hbm.at[idxc.at[pl.ds(o, C)]], bufb.at[s], gsem.at[s]).wait()

        def fire_write(j, s):
            o = base + j * C
            pltpu.async_copy(bufo.at[s], out_hbm.at[pl.ds(o, C)], wsem.at[s])

        def wait_write(j, s):
            o = base + j * C
            pltpu.make_async_copy(
                bufo.at[s], out_hbm.at[pl.ds(o, C)], wsem.at[s]).wait()

        himask = jnp.full((16,), -0x10000, dtype=jnp.int32)

        def widen_lo(v):
            return plsc.bitcast(v << 16, jnp.float32)

        def widen_hi(v):
            return plsc.bitcast(v & himask, jnp.float32)

        def do_add(s):
            @plsc.parallel_loop(0, C, unroll=4)
            def _add_body(e):
                for g in range(Dw // 16):
                    sl = pl.ds(g * 16, 16)
                    va = bufa[s, e, sl]
                    vb = bufb[s, e, sl]
                    bufo[s, e, pl.ds(g * 32, 16)] = widen_lo(va) + widen_lo(vb)
                    bufo[s, e, pl.ds(g * 32 + 16, 16)] = widen_hi(va) + widen_hi(vb)

        fire_gather(0, 0)
        fire_gather(1, 1)

        def outer(jj, carry):
            for s in range(NBUF):
                j = jj * NBUF + s
                if s < 2:
                    @pl.when(jj >= 1)
                    def _w():
                        wait_write(j - 2, (s + 2) % NBUF)
                else:
                    wait_write(j - 2, s - 2)
                fire_gather(j + 2, (s + 2) % NBUF)
                wait_gather(j, s)
                do_add(s)
                fire_write(j, s)
            return carry

        lax.fori_loop(0, nouter, outer, 0, unroll=False)

        for t in range(ntail):
            j = main + t
            wait_write(j - 2, (t + 2) % NBUF)
            wait_gather(j, t)
            do_add(t)
            fire_write(j, t)
        for t in range(ntail):
            wait_write(main + t, t)

    return sc_fn


def kernel(x, edge_index, W, b):
    n, d_in = x.shape
    d_out = W.shape[1]
    E = edge_index.shape[1]

    p = jnp.arange(d_out)
    perm = 32 * (p // 32) + (p % 32) // 2 + 16 * (p % 2)
    b_half = (0.5 * b)[perm].reshape(1, d_out).astype(jnp.float32)
    y = _tc_matmul_bf16(x, W[:, perm], b_half)
    y32 = lax.bitcast_convert_type(y.reshape(n, d_out // 2, 2), jnp.int32)

    sc_fn = _make_sc_gather_add(E, d_out, C=40, NBUF=4)
    row = edge_index[0]
    col = edge_index[1]
    return sc_fn(y32, row, col)


# speedup vs baseline: 1.9904x; 1.2144x over previous
import functools

import numpy as np

import jax
import jax.numpy as jnp
from jax import lax
from jax.experimental import pallas as pl
from jax.experimental.pallas import tpu as pltpu
from jax.experimental.pallas import tpu_sc as plsc

_NC = 2
_NS = 16
_NW = _NC * _NS


def _tc_matmul_pack(x, W_ab, b_ab):
    n, d_in = x.shape
    d_out = W_ab.shape[1]
    dw = d_out // 2

    def body(x_ref, w_ref, b_ref, o_ref):
        y = (
            jnp.dot(x_ref[...], w_ref[...], preferred_element_type=jnp.float32)
            + b_ref[...]
        )
        ybits = lax.bitcast_convert_type(
            y.astype(jnp.bfloat16).astype(jnp.float32), jnp.int32
        )
        a = ybits[:, :dw]
        bb = ybits[:, dw:]
        o_ref[...] = lax.shift_right_logical(a, 16) | (bb & jnp.int32(-0x10000))

    return pl.pallas_call(
        body,
        out_shape=jax.ShapeDtypeStruct((n, dw), jnp.int32),
    )(x, W_ab, b_ab)


def _make_sc_gather_add(E, D, C, NBUF):
    epw = E // _NW
    nchunks = epw // C
    nouter = nchunks // NBUF
    main = nouter * NBUF
    ntail = nchunks - main
    assert ntail == 2 and NBUF >= 4 and epw % C == 0 and C % 8 == 0 and C <= 128
    Dw = D // 2
    mesh = plsc.VectorSubcoreMesh(core_axis_name="c", subcore_axis_name="s")

    @functools.partial(
        pl.kernel,
        mesh=mesh,
        compiler_params=pltpu.CompilerParams(
            needs_layout_passes=False, use_tc_tiling_on_sc=False),
        out_type=jax.ShapeDtypeStruct((E, D), jnp.float32),
        scratch_types=[
            pltpu.VMEM((epw,), jnp.int32),
            pltpu.VMEM((epw,), jnp.int32),
            pltpu.VMEM((NBUF, C, Dw), jnp.int32),
            pltpu.VMEM((NBUF, C, Dw), jnp.int32),
            pltpu.VMEM((NBUF, C, D), jnp.float32),
            pltpu.SemaphoreType.DMA((NBUF,)),
            pltpu.SemaphoreType.DMA((NBUF,)),
        ],
    )
    def sc_fn(y_hbm, ei_hbm, out_hbm,
              idxr, idxc, bufa, bufb, bufo, gsem, wsem):
        wid = lax.axis_index("s") * _NC + lax.axis_index("c")
        base = wid * epw

        pltpu.sync_copy(ei_hbm.at[0, pl.ds(base, epw)], idxr)
        pltpu.sync_copy(ei_hbm.at[1, pl.ds(base, epw)], idxc)

        def fire_gather(j, s):
            o = j * C
            pltpu.async_copy(y_hbm.at[idxr.at[pl.ds(o, C)]], bufa.at[s], gsem.at[s])
            pltpu.async_copy(y_hbm.at[idxc.at[pl.ds(o, C)]], bufb.at[s], gsem.at[s])

        def wait_gather(j, s):
            o = j * C
            pltpu.make_async_copy(
                y_hbm.at[idxr.at[pl.ds(o, C)]], bufa.at[s], gsem.at[s]).wait()
            pltpu.make_async_copy(
                y_hbm.at[idxc.at[pl.ds(o, C)]], bufb.at[s], gsem.at[s]).wait()

        def fire_write(j, s):
            o = base + j * C
            pltpu.async_copy(bufo.at[s], out_hbm.at[pl.ds(o, C)], wsem.at[s])

        def wait_write(j, s):
            o = base + j * C
            pltpu.make_async_copy(
                bufo.at[s], out_hbm.at[pl.ds(o, C)], wsem.at[s]).wait()

        himask = jnp.full((16,), -0x10000, dtype=jnp.int32)

        def widen_lo(v):
            return plsc.bitcast(v << 16, jnp.float32)

        def widen_hi(v):
            return plsc.bitcast(v & himask, jnp.float32)

        def do_add(s):
            @plsc.parallel_loop(0, C, unroll=4)
            def _add_body(e):
                for g in range(Dw // 16):
                    sl = pl.ds(g * 16, 16)
                    va = bufa[s, e, sl]
                    vb = bufb[s, e, sl]
                    bufo[s, e, pl.ds(g * 32, 16)] = widen_lo(va) + widen_lo(vb)
                    bufo[s, e, pl.ds(g * 32 + 16, 16)] = widen_hi(va) + widen_hi(vb)

        fire_gather(0, 0)
        fire_gather(1, 1)

        def outer(jj, carry):
            for s in range(NBUF):
                j = jj * NBUF + s
                if s < 2:
                    @pl.when(jj >= 1)
                    def _w():
                        wait_write(j - 2, (s + 2) % NBUF)
                else:
                    wait_write(j - 2, s - 2)
                fire_gather(j + 2, (s + 2) % NBUF)
                wait_gather(j, s)
                do_add(s)
                fire_write(j, s)
            return carry

        lax.fori_loop(0, nouter, outer, 0, unroll=False)

        for t in range(ntail):
            j = main + t
            wait_write(j - 2, (t + 2) % NBUF)
            wait_gather(j, t)
            do_add(t)
            fire_write(j, t)
        for t in range(ntail):
            wait_write(main + t, t)

    return sc_fn


def kernel(x, edge_index, W, b):
    n, d_in = x.shape
    d_out = W.shape[1]
    E = edge_index.shape[1]

    cols_a = np.array([32 * g + k for g in range(d_out // 32) for k in range(16)])
    perm_ab = np.concatenate([cols_a, cols_a + 16])
    b_ab = (0.5 * b)[perm_ab].reshape(1, d_out).astype(jnp.float32)
    y32 = _tc_matmul_pack(x, W[:, perm_ab], b_ab)

    sc_fn = _make_sc_gather_add(E, d_out, C=40, NBUF=4)
    return sc_fn(y32, edge_index)


# speedup vs baseline: 2.1202x; 1.0652x over previous
import functools

import numpy as np

import jax
import jax.numpy as jnp
from jax import lax
from jax.experimental import pallas as pl
from jax.experimental.pallas import tpu as pltpu
from jax.experimental.pallas import tpu_sc as plsc

_NC = 2
_NS = 16
_NW = _NC * _NS


def _tc_matmul_pack(x, W_ab, b_ab):
    n, d_in = x.shape
    d_out = W_ab.shape[1]
    dw = d_out // 2

    def body(x_ref, w_ref, b_ref, o_ref):
        y = (
            jnp.dot(x_ref[...], w_ref[...], preferred_element_type=jnp.float32)
            + b_ref[...]
        )
        ybits = lax.bitcast_convert_type(
            y.astype(jnp.bfloat16).astype(jnp.float32), jnp.int32
        )
        a = ybits[:, :dw]
        bb = ybits[:, dw:]
        o_ref[...] = lax.shift_right_logical(a, 16) | (bb & jnp.int32(-0x10000))

    return pl.pallas_call(
        body,
        out_shape=jax.ShapeDtypeStruct((n, dw), jnp.int32),
    )(x, W_ab, b_ab)


def _make_sc_gather_add(E, D, C, NBUF):
    epw = E // _NW
    nchunks = epw // C
    nouter = nchunks // NBUF
    assert nchunks == nouter * NBUF
    assert NBUF >= 4 and epw % C == 0 and C % 8 == 0 and C <= 128
    Dw = D // 2
    mesh = plsc.VectorSubcoreMesh(core_axis_name="c", subcore_axis_name="s")

    @functools.partial(
        pl.kernel,
        mesh=mesh,
        compiler_params=pltpu.CompilerParams(
            needs_layout_passes=False, use_tc_tiling_on_sc=False),
        out_type=jax.ShapeDtypeStruct((E, D), jnp.float32),
        scratch_types=[
            pltpu.VMEM((epw,), jnp.int32),
            pltpu.VMEM((epw,), jnp.int32),
            pltpu.VMEM((NBUF, C, Dw), jnp.int32),
            pltpu.VMEM((NBUF, C, Dw), jnp.int32),
            pltpu.VMEM((NBUF, C, D), jnp.float32),
            pltpu.SemaphoreType.DMA((NBUF,)),
            pltpu.SemaphoreType.DMA((NBUF,)),
        ],
    )
    def sc_fn(y_hbm, ei_hbm, out_hbm,
              idxr, idxc, bufa, bufb, bufo, gsem, wsem):
        wid = lax.axis_index("s") * _NC + lax.axis_index("c")
        base = wid * epw

        pltpu.sync_copy(ei_hbm.at[0, pl.ds(base, epw)], idxr)
        pltpu.sync_copy(ei_hbm.at[1, pl.ds(base, epw)], idxc)

        def fire_gather(j, s):
            o = j * C
            pltpu.async_copy(y_hbm.at[idxr.at[pl.ds(o, C)]], bufa.at[s], gsem.at[s])
            pltpu.async_copy(y_hbm.at[idxc.at[pl.ds(o, C)]], bufb.at[s], gsem.at[s])

        def wait_gather(j, s):
            o = j * C
            pltpu.make_async_copy(
                y_hbm.at[idxr.at[pl.ds(o, C)]], bufa.at[s], gsem.at[s]).wait()
            pltpu.make_async_copy(
                y_hbm.at[idxc.at[pl.ds(o, C)]], bufb.at[s], gsem.at[s]).wait()

        def fire_write(j, s):
            o = base + j * C
            pltpu.async_copy(bufo.at[s], out_hbm.at[pl.ds(o, C)], wsem.at[s])

        def wait_write(j, s):
            o = base + j * C
            pltpu.make_async_copy(
                bufo.at[s], out_hbm.at[pl.ds(o, C)], wsem.at[s]).wait()

        himask = jnp.full((16,), -0x10000, dtype=jnp.int32)

        def widen_lo(v):
            return plsc.bitcast(v << 16, jnp.float32)

        def widen_hi(v):
            return plsc.bitcast(v & himask, jnp.float32)

        def do_add(s):
            @plsc.parallel_loop(0, C, unroll=4)
            def _add_body(e):
                for g in range(Dw // 16):
                    sl = pl.ds(g * 16, 16)
                    va = bufa[s, e, sl]
                    vb = bufb[s, e, sl]
                    bufo[s, e, pl.ds(g * 32, 16)] = widen_lo(va) + widen_lo(vb)
                    bufo[s, e, pl.ds(g * 32 + 16, 16)] = widen_hi(va) + widen_hi(vb)

        fire_gather(0, 0)
        fire_gather(1, 1)

        def outer(jj, carry):
            for s in range(NBUF):
                j = jj * NBUF + s
                if s < 2:
                    @pl.when(jj >= 1)
                    def _w():
                        wait_write(j - 2, (s - 2) % NBUF)
                else:
                    wait_write(j - 2, s - 2)
                if s < NBUF - 2:
                    fire_gather(j + 2, (s + 2) % NBUF)
                else:
                    @pl.when(jj < nouter - 1)
                    def _g():
                        fire_gather(j + 2, (s + 2) % NBUF)
                wait_gather(j, s)
                do_add(s)
                fire_write(j, s)
            return carry

        lax.fori_loop(0, nouter, outer, 0, unroll=False)

        for t in (nchunks - 2, nchunks - 1):
            wait_write(t, t % NBUF)

    return sc_fn


def kernel(x, edge_index, W, b):
    n, d_in = x.shape
    d_out = W.shape[1]
    E = edge_index.shape[1]

    cols_a = np.array([32 * g + k for g in range(d_out // 32) for k in range(16)])
    perm_ab = np.concatenate([cols_a, cols_a + 16])
    b_ab = (0.5 * b)[perm_ab].reshape(1, d_out).astype(jnp.float32)
    y32 = _tc_matmul_pack(x, W[:, perm_ab], b_ab)

    sc_fn = _make_sc_gather_add(E, d_out, C=80, NBUF=5)
    return sc_fn(y32, edge_index)
